# baseline (speedup 1.0000x reference)
import functools
import math

import jax
import jax.numpy as jnp
import numpy as np
from jax.experimental import pallas as pl
from jax.experimental.pallas import tpu as pltpu

_EF = 128


def _linear_kernel(x_ref, w_ref, b_ref, o_ref, *, relu):
    y = jnp.dot(x_ref[...], w_ref[...], preferred_element_type=jnp.float32)
    y = y + b_ref[...]
    if relu:
        y = jnp.maximum(y, 0.0)
    o_ref[...] = y


def _plin(p, x, relu=False):
    w, b = p["W"], p["b"]
    din, dout = w.shape
    shp = x.shape
    x2 = x.reshape(-1, din)
    r = x2.shape[0]
    tr = min(r, 512)
    out = pl.pallas_call(
        functools.partial(_linear_kernel, relu=relu),
        grid=(r // tr,),
        in_specs=[
            pl.BlockSpec((tr, din), lambda i: (i, 0)),
            pl.BlockSpec((din, dout), lambda i: (0, 0)),
            pl.BlockSpec((1, dout), lambda i: (0, 0)),
        ],
        out_specs=pl.BlockSpec((tr, dout), lambda i: (i, 0)),
        out_shape=jax.ShapeDtypeStruct((r, dout), jnp.float32),
    )(x2, w, b.reshape(1, -1))
    return out.reshape(*shp[:-1], dout)


def _dist_kernel(q_ref, st_ref, o_ref):
    q = q_ref[0]
    d = None
    for a in range(3):
        qa = q[:, a : a + 1]
        sa = st_ref[0, a : a + 1, :]
        diff = qa - sa
        d = diff * diff if d is None else d + diff * diff
    o_ref[0] = d


def _pdist(query, src):
    b, nq, _ = query.shape
    ns = src.shape[1]
    st = jnp.transpose(src, (0, 2, 1))
    tq = 512
    return pl.pallas_call(
        _dist_kernel,
        grid=(b, nq // tq),
        in_specs=[
            pl.BlockSpec((1, tq, 3), lambda bi, i: (bi, i, 0)),
            pl.BlockSpec((1, 3, ns), lambda bi, i: (bi, 0, 0)),
        ],
        out_specs=pl.BlockSpec((1, tq, ns), lambda bi, i: (bi, i, 0)),
        out_shape=jax.ShapeDtypeStruct((b, nq, ns), jnp.float32),
    )(query, st)


_OMEGA = (1.0 / (10000.0 ** (np.arange(10, dtype=np.float32) / np.float32(10.0))))


def _peg_kernel(om_ref, pos_ref, gqm_ref, gv_ref, w0_ref, b0_ref, w1_ref,
                b1_ref, g_ref, gf_ref):
    x = pos_ref[0]
    om = om_ref[...]
    pieces = []
    for a in range(3):
        out = x[:, a : a + 1] * om
        pieces.append(jnp.sin(out))
        pieces.append(jnp.cos(out))
    emb = jnp.concatenate(pieces, axis=-1)
    h = jnp.dot(emb, w0_ref[...], preferred_element_type=jnp.float32)
    h = jnp.maximum(h + b0_ref[...], 0.0)
    pe = jnp.dot(h, w1_ref[...], preferred_element_type=jnp.float32)
    pe = pe + b1_ref[...]
    g_ref[0] = gqm_ref[0] + pe
    gf_ref[0] = gv_ref[0] + pe


def _attn_kernel(g_ref, wr_ref, br_ref, gf_ref, o_ref, *, jc, dout):
    j = pl.program_id(2)
    w = jnp.dot(g_ref[0], wr_ref[...], preferred_element_type=jnp.float32)
    w = w + br_ref[...]
    tn = w.shape[0]
    w = w.reshape(tn, jc, dout)
    s = jnp.sum(jnp.abs(w) + 1e-07, axis=-1, keepdims=True)
    w = w / s * math.sqrt(dout)
    gf = gf_ref[0]
    acc = gf[0] * w[:, 0, :]
    for jj in range(1, jc):
        acc = acc + gf[jj] * w[:, jj, :]

    @pl.when(j == 0)
    def _init():
        o_ref[0] = acc

    @pl.when(j > 0)
    def _accum():
        o_ref[0] = o_ref[0] + acc


def _transformer_layer(p, feature, xyz, dout):
    b, n, _ = feature.shape
    d = _pdist(xyz, xyz)
    idx = jax.lax.top_k(-d, 36)[1]
    q = _plin(p["linear_q"], feature, relu=True)
    v = _plin(p["linear_v"], feature, relu=True)
    gat = jax.vmap(lambda a, i: a[i])
    gqm = gat(q, idx) - q[:, :, None, :]
    gv = gat(v, idx)
    pos = gat(xyz, idx) - xyz[:, :, None, :]

    k36 = n * 36
    tn = 128
    t36 = tn * 36
    dp = p["fc_delta0"]
    dp1 = p["fc_delta1"]
    g36, gff = pl.pallas_call(
        _peg_kernel,
        grid=(b, n // tn),
        in_specs=[
            pl.BlockSpec((1, 10), lambda bi, i: (0, 0)),
            pl.BlockSpec((1, t36, 3), lambda bi, i: (bi, i, 0)),
            pl.BlockSpec((1, t36, dout), lambda bi, i: (bi, i, 0)),
            pl.BlockSpec((1, t36, dout), lambda bi, i: (bi, i, 0)),
            pl.BlockSpec((60, dout), lambda bi, i: (0, 0)),
            pl.BlockSpec((1, dout), lambda bi, i: (0, 0)),
            pl.BlockSpec((dout, dout), lambda bi, i: (0, 0)),
            pl.BlockSpec((1, dout), lambda bi, i: (0, 0)),
        ],
        out_specs=[
            pl.BlockSpec((1, t36, dout), lambda bi, i: (bi, i, 0)),
            pl.BlockSpec((1, t36, dout), lambda bi, i: (bi, i, 0)),
        ],
        out_shape=[
            jax.ShapeDtypeStruct((b, k36, dout), jnp.float32),
            jax.ShapeDtypeStruct((b, k36, dout), jnp.float32),
        ],
    )(
        jnp.asarray(_OMEGA).reshape(1, 10),
        pos.reshape(b, k36, 3),
        gqm.reshape(b, k36, dout),
        gv.reshape(b, k36, dout),
        dp["W"], dp["b"].reshape(1, -1), dp1["W"], dp1["b"].reshape(1, -1),
    )

    g = g36.reshape(b, n, 36 * dout)
    gf4 = jnp.transpose(gff.reshape(b, n, 36, dout), (0, 2, 1, 3))
    k = 36 * dout
    jc = 36 if dout <= 32 else 4
    wr = p["linear_r"]["W"]
    br = p["linear_r"]["b"].reshape(1, -1)
    feat = pl.pallas_call(
        functools.partial(_attn_kernel, jc=jc, dout=dout),
        grid=(b, n // tn, 36 // jc),
        in_specs=[
            pl.BlockSpec((1, tn, k), lambda bi, i, j: (bi, i, 0)),
            pl.BlockSpec((k, jc * dout), lambda bi, i, j: (0, j)),
            pl.BlockSpec((1, jc * dout), lambda bi, i, j: (0, j)),
            pl.BlockSpec((1, jc, tn, dout), lambda bi, i, j: (bi, j, i, 0)),
        ],
        out_specs=pl.BlockSpec((1, tn, dout), lambda bi, i, j: (bi, i, 0)),
        out_shape=jax.ShapeDtypeStruct((b, n, dout), jnp.float32),
        compiler_params=pltpu.CompilerParams(
            dimension_semantics=("parallel", "parallel", "arbitrary")
        ),
    )(g, wr, br, gf4)
    return _plin(p["suffix"], feat)


def _bn_kernel(x_ref, g_ref, b_ref, o_ref):
    x = x_ref[...]
    m = jnp.mean(x, axis=0, keepdims=True)
    v = jnp.mean((x - m) ** 2, axis=0, keepdims=True)
    o_ref[...] = (x - m) / jnp.sqrt(v + 1e-05) * g_ref[...] + b_ref[...]


def _bn(p, x):
    b, n, c = x.shape
    out = pl.pallas_call(
        _bn_kernel,
        in_specs=[
            pl.BlockSpec((b * n, c), lambda: (0, 0)),
            pl.BlockSpec((1, c), lambda: (0, 0)),
            pl.BlockSpec((1, c), lambda: (0, 0)),
        ],
        out_specs=pl.BlockSpec((b * n, c), lambda: (0, 0)),
        out_shape=jax.ShapeDtypeStruct((b * n, c), jnp.float32),
    )(x.reshape(b * n, c), p["gamma"].reshape(1, -1), p["beta"].reshape(1, -1))
    return out.reshape(b, n, c)


def _ind_kernel(gx_ref, gf_ref, w0_ref, b0_ref, w1_ref, b1_ref, o_ref, *, knn):
    x = gx_ref[0]
    h = jnp.dot(x, w0_ref[...], preferred_element_type=jnp.float32)
    h = jnp.maximum(h + b0_ref[...], 0.0)
    pw = jnp.dot(h, w1_ref[...], preferred_element_type=jnp.float32)
    pw = pw + b1_ref[...]
    prod = pw * gf_ref[0]
    tk, c = prod.shape
    red = jnp.sum(prod.reshape(tk // knn, knn, c), axis=1)
    o_ref[0] = red * (1.0 / math.sqrt(knn))


def kernel(xyz, detect_point, normal_gt, params):
    p = params
    ef = _EF
    f1 = _bn(p["bn1"], _transformer_layer(p["tl1"], xyz, xyz, ef // 4))
    f2 = _bn(p["bn2"], _transformer_layer(p["tl2"], f1, xyz, ef))
    f3 = _bn(p["bn3"], _transformer_layer(p["tl3"], f2, xyz, ef))
    f4 = _bn(p["bn4"], _transformer_layer(p["tl4"], f3, xyz, ef))
    f4 = f4 + _plin(p["res_fc"], f2)

    knn = 12
    fpre = _plin(p["ind_prefix"], f4)
    d = _pdist(detect_point, xyz)
    neg, idx = jax.lax.top_k(-d, knn)
    min_d = -neg[:, :, 0]
    weight = jnp.where(min_d > 0.03, 10.0, 1.0)

    gat = jax.vmap(lambda a, i: a[i])
    gf = gat(fpre, idx)
    gx = gat(xyz, idx) - detect_point[:, :, None, :]

    b, m, _ = detect_point.shape
    c = 4 * ef
    tm = 256
    tk = tm * knn
    p0, p1 = p["ind_pos0"], p["ind_pos1"]
    nf = pl.pallas_call(
        functools.partial(_ind_kernel, knn=knn),
        grid=(b, m // tm),
        in_specs=[
            pl.BlockSpec((1, tk, 3), lambda bi, i: (bi, i, 0)),
            pl.BlockSpec((1, tk, c), lambda bi, i: (bi, i, 0)),
            pl.BlockSpec((3, c), lambda bi, i: (0, 0)),
            pl.BlockSpec((1, c), lambda bi, i: (0, 0)),
            pl.BlockSpec((c, c), lambda bi, i: (0, 0)),
            pl.BlockSpec((1, c), lambda bi, i: (0, 0)),
        ],
        out_specs=pl.BlockSpec((1, tm, c), lambda bi, i: (bi, i, 0)),
        out_shape=jax.ShapeDtypeStruct((b, m, c), jnp.float32),
    )(
        gx.reshape(b, m * knn, 3),
        gf.reshape(b, m * knn, c),
        p0["W"], p0["b"].reshape(1, -1), p1["W"], p1["b"].reshape(1, -1),
    )

    h = _plin(p["cls0"], nf, relu=True)
    occ = _plin(p["cls1"], h)
    return occ, weight


# speedup vs baseline: 1.0459x; 1.0459x over previous
import functools
import math

import jax
import jax.numpy as jnp
import numpy as np
from jax.experimental import pallas as pl
from jax.experimental.pallas import tpu as pltpu

_EF = 128


def _linear_kernel(x_ref, w_ref, b_ref, o_ref, *, relu):
    y = jnp.dot(x_ref[...], w_ref[...], preferred_element_type=jnp.float32)
    y = y + b_ref[...]
    if relu:
        y = jnp.maximum(y, 0.0)
    o_ref[...] = y


def _plin(p, x, relu=False):
    w, b = p["W"], p["b"]
    din, dout = w.shape
    shp = x.shape
    x2 = x.reshape(-1, din)
    r = x2.shape[0]
    tr = min(r, 512)
    out = pl.pallas_call(
        functools.partial(_linear_kernel, relu=relu),
        grid=(r // tr,),
        in_specs=[
            pl.BlockSpec((tr, din), lambda i: (i, 0)),
            pl.BlockSpec((din, dout), lambda i: (0, 0)),
            pl.BlockSpec((1, dout), lambda i: (0, 0)),
        ],
        out_specs=pl.BlockSpec((tr, dout), lambda i: (i, 0)),
        out_shape=jax.ShapeDtypeStruct((r, dout), jnp.float32),
    )(x2, w, b.reshape(1, -1))
    return out.reshape(*shp[:-1], dout)


def _dist_kernel(q_ref, st_ref, o_ref):
    q = q_ref[0]
    d = None
    for a in range(3):
        qa = q[:, a : a + 1]
        sa = st_ref[0, a : a + 1, :]
        diff = qa - sa
        d = diff * diff if d is None else d + diff * diff
    o_ref[0] = d


def _pdist(query, src):
    b, nq, _ = query.shape
    ns = src.shape[1]
    st = jnp.transpose(src, (0, 2, 1))
    tq = 512
    return pl.pallas_call(
        _dist_kernel,
        grid=(b, nq // tq),
        in_specs=[
            pl.BlockSpec((1, tq, 3), lambda bi, i: (bi, i, 0)),
            pl.BlockSpec((1, 3, ns), lambda bi, i: (bi, 0, 0)),
        ],
        out_specs=pl.BlockSpec((1, tq, ns), lambda bi, i: (bi, i, 0)),
        out_shape=jax.ShapeDtypeStruct((b, nq, ns), jnp.float32),
    )(query, st)


_OMEGA = (1.0 / (10000.0 ** (np.arange(10, dtype=np.float32) / np.float32(10.0))))


def _peg_kernel(om_ref, pos_ref, gqm_ref, gv_ref, w0_ref, b0_ref, w1_ref,
                b1_ref, g_ref, gf_ref):
    x = pos_ref[0]
    om = om_ref[...]
    pieces = []
    for a in range(3):
        out = x[:, a : a + 1] * om
        pieces.append(jnp.sin(out))
        pieces.append(jnp.cos(out))
    emb = jnp.concatenate(pieces, axis=-1)
    h = jnp.dot(emb, w0_ref[...], preferred_element_type=jnp.float32)
    h = jnp.maximum(h + b0_ref[...], 0.0)
    pe = jnp.dot(h, w1_ref[...], preferred_element_type=jnp.float32)
    pe = pe + b1_ref[...]
    g_ref[0] = gqm_ref[0] + pe
    gf_ref[0] = gv_ref[0] + pe


def _attn_kernel(g_ref, wr_ref, br_ref, gf_ref, o_ref, *, jc, dout):
    j = pl.program_id(2)
    w = jnp.dot(g_ref[0], wr_ref[...], preferred_element_type=jnp.float32)
    w = w + br_ref[...]
    tn = w.shape[0]
    w = w.reshape(tn, jc, dout)
    s = jnp.sum(jnp.abs(w) + 1e-07, axis=-1, keepdims=True)
    w = w / s * math.sqrt(dout)
    gf = gf_ref[0]
    acc = gf[0] * w[:, 0, :]
    for jj in range(1, jc):
        acc = acc + gf[jj] * w[:, jj, :]

    @pl.when(j == 0)
    def _init():
        o_ref[0] = acc

    @pl.when(j > 0)
    def _accum():
        o_ref[0] = o_ref[0] + acc


def _transformer_layer(p, feature, xyz, dout):
    b, n, _ = feature.shape
    d = _pdist(xyz, xyz)
    idx = jax.lax.top_k(-d, 36)[1]
    q = _plin(p["linear_q"], feature, relu=True)
    v = _plin(p["linear_v"], feature, relu=True)
    gat = jax.vmap(lambda a, i: a[i])
    gqm = gat(q, idx) - q[:, :, None, :]
    gv = gat(v, idx)
    pos = gat(xyz, idx) - xyz[:, :, None, :]

    k36 = n * 36
    tn = 128
    t36 = tn * 36
    dp = p["fc_delta0"]
    dp1 = p["fc_delta1"]
    g36, gff = pl.pallas_call(
        _peg_kernel,
        grid=(b, n // tn),
        in_specs=[
            pl.BlockSpec((1, 10), lambda bi, i: (0, 0)),
            pl.BlockSpec((1, t36, 3), lambda bi, i: (bi, i, 0)),
            pl.BlockSpec((1, t36, dout), lambda bi, i: (bi, i, 0)),
            pl.BlockSpec((1, t36, dout), lambda bi, i: (bi, i, 0)),
            pl.BlockSpec((60, dout), lambda bi, i: (0, 0)),
            pl.BlockSpec((1, dout), lambda bi, i: (0, 0)),
            pl.BlockSpec((dout, dout), lambda bi, i: (0, 0)),
            pl.BlockSpec((1, dout), lambda bi, i: (0, 0)),
        ],
        out_specs=[
            pl.BlockSpec((1, t36, dout), lambda bi, i: (bi, i, 0)),
            pl.BlockSpec((1, t36, dout), lambda bi, i: (bi, i, 0)),
        ],
        out_shape=[
            jax.ShapeDtypeStruct((b, k36, dout), jnp.float32),
            jax.ShapeDtypeStruct((b, k36, dout), jnp.float32),
        ],
    )(
        jnp.asarray(_OMEGA).reshape(1, 10),
        pos.reshape(b, k36, 3),
        gqm.reshape(b, k36, dout),
        gv.reshape(b, k36, dout),
        dp["W"], dp["b"].reshape(1, -1), dp1["W"], dp1["b"].reshape(1, -1),
    )

    g = g36.reshape(b, n, 36 * dout)
    gf4 = jnp.transpose(gff.reshape(b, n, 36, dout), (0, 2, 1, 3))
    k = 36 * dout
    jc = 36 if dout <= 32 else 4
    tna = 256 if dout <= 32 else 512
    wr = p["linear_r"]["W"]
    br = p["linear_r"]["b"].reshape(1, -1)
    feat = pl.pallas_call(
        functools.partial(_attn_kernel, jc=jc, dout=dout),
        grid=(b, n // tna, 36 // jc),
        in_specs=[
            pl.BlockSpec((1, tna, k), lambda bi, i, j: (bi, i, 0)),
            pl.BlockSpec((k, jc * dout), lambda bi, i, j: (0, j)),
            pl.BlockSpec((1, jc * dout), lambda bi, i, j: (0, j)),
            pl.BlockSpec((1, jc, tna, dout), lambda bi, i, j: (bi, j, i, 0)),
        ],
        out_specs=pl.BlockSpec((1, tna, dout), lambda bi, i, j: (bi, i, 0)),
        out_shape=jax.ShapeDtypeStruct((b, n, dout), jnp.float32),
        compiler_params=pltpu.CompilerParams(
            dimension_semantics=("parallel", "parallel", "arbitrary")
        ),
    )(g, wr, br, gf4)
    return _plin(p["suffix"], feat)


def _bn_kernel(x_ref, g_ref, b_ref, o_ref):
    x = x_ref[...]
    m = jnp.mean(x, axis=0, keepdims=True)
    v = jnp.mean((x - m) ** 2, axis=0, keepdims=True)
    o_ref[...] = (x - m) / jnp.sqrt(v + 1e-05) * g_ref[...] + b_ref[...]


def _bn(p, x):
    b, n, c = x.shape
    out = pl.pallas_call(
        _bn_kernel,
        in_specs=[
            pl.BlockSpec((b * n, c), lambda: (0, 0)),
            pl.BlockSpec((1, c), lambda: (0, 0)),
            pl.BlockSpec((1, c), lambda: (0, 0)),
        ],
        out_specs=pl.BlockSpec((b * n, c), lambda: (0, 0)),
        out_shape=jax.ShapeDtypeStruct((b * n, c), jnp.float32),
    )(x.reshape(b * n, c), p["gamma"].reshape(1, -1), p["beta"].reshape(1, -1))
    return out.reshape(b, n, c)


def _ind_kernel(gx_ref, gf_ref, w0_ref, b0_ref, w1_ref, b1_ref, o_ref, *, knn):
    x = gx_ref[0]
    h = jnp.dot(x, w0_ref[...], preferred_element_type=jnp.float32)
    h = jnp.maximum(h + b0_ref[...], 0.0)
    pw = jnp.dot(h, w1_ref[...], preferred_element_type=jnp.float32)
    pw = pw + b1_ref[...]
    prod = pw * gf_ref[0]
    tk, c = prod.shape
    red = jnp.sum(prod.reshape(tk // knn, knn, c), axis=1)
    o_ref[0] = red * (1.0 / math.sqrt(knn))


def kernel(xyz, detect_point, normal_gt, params):
    p = params
    ef = _EF
    f1 = _bn(p["bn1"], _transformer_layer(p["tl1"], xyz, xyz, ef // 4))
    f2 = _bn(p["bn2"], _transformer_layer(p["tl2"], f1, xyz, ef))
    f3 = _bn(p["bn3"], _transformer_layer(p["tl3"], f2, xyz, ef))
    f4 = _bn(p["bn4"], _transformer_layer(p["tl4"], f3, xyz, ef))
    f4 = f4 + _plin(p["res_fc"], f2)

    knn = 12
    fpre = _plin(p["ind_prefix"], f4)
    d = _pdist(detect_point, xyz)
    neg, idx = jax.lax.top_k(-d, knn)
    min_d = -neg[:, :, 0]
    weight = jnp.where(min_d > 0.03, 10.0, 1.0)

    gat = jax.vmap(lambda a, i: a[i])
    gf = gat(fpre, idx)
    gx = gat(xyz, idx) - detect_point[:, :, None, :]

    b, m, _ = detect_point.shape
    c = 4 * ef
    tm = 256
    tk = tm * knn
    p0, p1 = p["ind_pos0"], p["ind_pos1"]
    nf = pl.pallas_call(
        functools.partial(_ind_kernel, knn=knn),
        grid=(b, m // tm),
        in_specs=[
            pl.BlockSpec((1, tk, 3), lambda bi, i: (bi, i, 0)),
            pl.BlockSpec((1, tk, c), lambda bi, i: (bi, i, 0)),
            pl.BlockSpec((3, c), lambda bi, i: (0, 0)),
            pl.BlockSpec((1, c), lambda bi, i: (0, 0)),
            pl.BlockSpec((c, c), lambda bi, i: (0, 0)),
            pl.BlockSpec((1, c), lambda bi, i: (0, 0)),
        ],
        out_specs=pl.BlockSpec((1, tm, c), lambda bi, i: (bi, i, 0)),
        out_shape=jax.ShapeDtypeStruct((b, m, c), jnp.float32),
    )(
        gx.reshape(b, m * knn, 3),
        gf.reshape(b, m * knn, c),
        p0["W"], p0["b"].reshape(1, -1), p1["W"], p1["b"].reshape(1, -1),
    )

    h = _plin(p["cls0"], nf, relu=True)
    occ = _plin(p["cls1"], h)
    return occ, weight
